# baseline (speedup 1.0000x reference)
import functools

import jax
import jax.numpy as jnp
import numpy as np
from jax import lax
from jax.experimental import pallas as pl
from jax.experimental.pallas import tpu as pltpu
from jax.experimental.pallas import tpu_sc as plsc

N = 10000
E = 320000
D = 128
ED = 16
H = D // 2

NC = 2
NS = 16
NW = NC * NS
EPW = E // NW
C = 40
NCHUNK = EPW // C
ZCH = 40
NZ = N // ZCH

_HI = np.int32(-65536)


def _pack2(lo_f32, hi_f32):
    lob = lax.bitcast_convert_type(
        lo_f32.astype(jnp.bfloat16).astype(jnp.float32), jnp.uint32)
    hib = lax.bitcast_convert_type(
        hi_f32.astype(jnp.bfloat16).astype(jnp.float32), jnp.uint32)
    return lax.bitcast_convert_type((lob >> 16) | (hib & jnp.uint32(0xFFFF0000)),
                                    jnp.int32)


def _g_kernel(x_ref, w1t_ref, g_ref):
    p = -jnp.dot(x_ref[...], w1t_ref[...], preferred_element_type=jnp.float32)
    g_ref[...] = _pack2(x_ref[...], p)


def _q_kernel(ea_ref, w2t_ref, b_ref, o_ref):
    q = -(jnp.dot(ea_ref[...], w2t_ref[...],
                  preferred_element_type=jnp.float32) + b_ref[...])
    o_ref[...] = _pack2(q[:, :H], q[:, H:])


def _out_kernel(x_ref, a0_ref, a1_ref, wnt_ref, wwt_ref, b_ref, o_ref):
    acc = jnp.dot(x_ref[...], wnt_ref[...], preferred_element_type=jnp.float32)
    aggr = a0_ref[...] + a1_ref[...]
    acc += jnp.dot(aggr, wwt_ref[...], preferred_element_type=jnp.float32)
    o_ref[...] = jnp.tanh(acc + b_ref[...])


def _lo_f32(w):
    return lax.bitcast_convert_type(lax.shift_left(w, 16), jnp.float32)


def _hi_f32(w):
    return lax.bitcast_convert_type(lax.bitwise_and(w, _HI), jnp.float32)


def _sc_body(g_hbm, q_hbm, col_hbm, row_hbm, out_hbm,
             colv0, colv1, rowv0, rowv1, gv0, gv1, qv0, qv1, vv,
             zbuf, aggr_sh, gsem0, gsem1, qsem0, qsem1):
    c = lax.axis_index("c")
    s = lax.axis_index("s")
    wid = c * NS + s
    ebase = wid * EPW
    colv = (colv0, colv1)
    rowv = (rowv0, rowv1)
    gv = (gv0, gv1)
    qv = (qv0, qv1)
    gsem = (gsem0, gsem1)
    qsem = (qsem0, qsem1)

    @plsc.parallel_loop(0, ZCH, unroll=4)
    def _zero_row(r):
        for g in range(D // 16):
            zbuf[r, pl.ds(16 * g, 16)] = jnp.zeros((16,), jnp.float32)
    for k in range(pl.cdiv(NZ, NS)):
        t = s + k * NS
        if (k + 1) * NS <= NZ:
            pltpu.sync_copy(zbuf, aggr_sh.at[pl.ds(t * ZCH, ZCH)])
        else:
            @pl.when(t < NZ)
            def _():
                pltpu.sync_copy(zbuf, aggr_sh.at[pl.ds(t * ZCH, ZCH)])
    plsc.subcore_barrier()

    def _start_loads(j, b):
        base = ebase + j * C
        pltpu.sync_copy(col_hbm.at[pl.ds(base, C)], colv[b])
        pltpu.sync_copy(row_hbm.at[pl.ds(base, C)], rowv[b])
        pltpu.async_copy(g_hbm.at[colv[b]], gv[b], gsem[b])
        pltpu.async_copy(q_hbm.at[pl.ds(base, C)], qv[b], qsem[b])

    def _round(i, _):
        for b in range(2):
            j = 2 * i + b
            _start_loads(j, b)
            pltpu.make_async_copy(g_hbm.at[colv[b]], gv[b], gsem[b]).wait()
            pltpu.make_async_copy(q_hbm.at[pl.ds(0, C)], qv[b], qsem[b]).wait()

            @plsc.parallel_loop(0, C, unroll=4)
            def _edge(r):
                for t in range(H // 16):
                    qw = qv[b][r, pl.ds(16 * t, 16)]
                    for half in range(2):
                        base = H * half + 16 * t
                        gw = gv[b][r, pl.ds(base, 16)]
                        xval = _lo_f32(gw)
                        pval = _hi_f32(gw)
                        qval = _lo_f32(qw) if half == 0 else _hi_f32(qw)
                        a = 1.0 / (1.0 + jnp.exp(pval + qval))
                        vv[r, pl.ds(base, 16)] = xval * a

            pltpu.sync_copy(vv, aggr_sh.at[rowv[b]], add=True)
        return _
    lax.fori_loop(0, NCHUNK // 2, _round, None)
    plsc.subcore_barrier()

    for k in range(pl.cdiv(NZ, NS)):
        t = s + k * NS

        def _copy_out(t=t):
            pltpu.sync_copy(aggr_sh.at[pl.ds(t * ZCH, ZCH)], zbuf)
            pltpu.sync_copy(zbuf, out_hbm.at[c, pl.ds(t * ZCH, ZCH)])
        if (k + 1) * NS <= NZ:
            _copy_out()
        else:
            pl.when(t < NZ)(_copy_out)


_sc_scatter = functools.partial(
    pl.kernel,
    out_type=jax.ShapeDtypeStruct((NC, N, D), jnp.float32),
    mesh=plsc.VectorSubcoreMesh(core_axis_name="c", subcore_axis_name="s"),
    scratch_types=[
        pltpu.VMEM((C,), jnp.int32),
        pltpu.VMEM((C,), jnp.int32),
        pltpu.VMEM((C,), jnp.int32),
        pltpu.VMEM((C,), jnp.int32),
        pltpu.VMEM((C, D), jnp.int32),
        pltpu.VMEM((C, D), jnp.int32),
        pltpu.VMEM((C, H), jnp.int32),
        pltpu.VMEM((C, H), jnp.int32),
        pltpu.VMEM((C, D), jnp.float32),
        pltpu.VMEM((ZCH, D), jnp.float32),
        pltpu.VMEM_SHARED((N, D), jnp.float32),
        pltpu.SemaphoreType.DMA,
        pltpu.SemaphoreType.DMA,
        pltpu.SemaphoreType.DMA,
        pltpu.SemaphoreType.DMA,
    ],
)(_sc_body)


def kernel(x, edge_index, edge_attr, W_node_w, W_node_b, W_neigh_w, W_neigh_b,
           W_att_w, W_att_b):
    row = edge_index[0]
    col = edge_index[1]
    W1t = W_att_w[:, :D].T
    W2t = W_att_w[:, D:].T

    G = pl.pallas_call(
        _g_kernel,
        out_shape=jax.ShapeDtypeStruct((N, D), jnp.int32),
    )(x, W1t)

    QB = 8000
    Q = pl.pallas_call(
        _q_kernel,
        grid=(E // QB,),
        in_specs=[
            pl.BlockSpec((QB, ED), lambda i: (i, 0)),
            pl.BlockSpec((ED, D), lambda i: (0, 0)),
            pl.BlockSpec((1, D), lambda i: (0, 0)),
        ],
        out_specs=pl.BlockSpec((QB, H), lambda i: (i, 0)),
        out_shape=jax.ShapeDtypeStruct((E, H), jnp.int32),
    )(edge_attr, W2t, W_att_b.reshape(1, D))

    aggr_parts = _sc_scatter(G, Q, col, row)

    out = pl.pallas_call(
        _out_kernel,
        out_shape=jax.ShapeDtypeStruct((N, D), jnp.float32),
    )(x, aggr_parts[0], aggr_parts[1], W_node_w.T, W_neigh_w.T,
      (W_node_b + W_neigh_b).reshape(1, D))
    return out


# speedup vs baseline: 1.3062x; 1.3062x over previous
import functools

import jax
import jax.numpy as jnp
import numpy as np
from jax import lax
from jax.experimental import pallas as pl
from jax.experimental.pallas import tpu as pltpu
from jax.experimental.pallas import tpu_sc as plsc

N = 10000
E = 320000
D = 128
ED = 16
H = D // 2

NC = 2
NS = 16
NW = NC * NS
EPW = E // NW
C = 40
NCHUNK = EPW // C
ZCH = 40
NZ = N // ZCH

_HI = np.int32(-65536)


def _pack2(lo_f32, hi_f32):
    lob = lax.bitcast_convert_type(
        lo_f32.astype(jnp.bfloat16).astype(jnp.float32), jnp.uint32)
    hib = lax.bitcast_convert_type(
        hi_f32.astype(jnp.bfloat16).astype(jnp.float32), jnp.uint32)
    return lax.bitcast_convert_type((lob >> 16) | (hib & jnp.uint32(0xFFFF0000)),
                                    jnp.int32)


def _g_kernel(x_ref, w1t_ref, g_ref):
    p = -jnp.dot(x_ref[...], w1t_ref[...], preferred_element_type=jnp.float32)
    g_ref[...] = _pack2(x_ref[...], p)


def _q_kernel(ea_ref, w2t_ref, b_ref, o_ref):
    q = -(jnp.dot(ea_ref[...], w2t_ref[...],
                  preferred_element_type=jnp.float32) + b_ref[...])
    o_ref[...] = _pack2(q[:, :H], q[:, H:])


def _out_kernel(x_ref, a0_ref, a1_ref, wnt_ref, wwt_ref, b_ref, o_ref):
    acc = jnp.dot(x_ref[...], wnt_ref[...], preferred_element_type=jnp.float32)
    aggr = a0_ref[...] + a1_ref[...]
    acc += jnp.dot(aggr, wwt_ref[...], preferred_element_type=jnp.float32)
    o_ref[...] = jnp.tanh(acc + b_ref[...])


def _lo_f32(w):
    return lax.bitcast_convert_type(lax.shift_left(w, 16), jnp.float32)


def _hi_f32(w):
    return lax.bitcast_convert_type(lax.bitwise_and(w, _HI), jnp.float32)


def _sc_body(g_hbm, q_hbm, col_hbm, row_hbm, out_hbm,
             colv0, colv1, rowv0, rowv1, gv0, gv1, qv0, qv1, vv,
             zbuf, aggr_sh, gsem0, gsem1, qsem0, qsem1):
    c = lax.axis_index("c")
    s = lax.axis_index("s")
    wid = c * NS + s
    ebase = wid * EPW
    colv = (colv0, colv1)
    rowv = (rowv0, rowv1)
    gv = (gv0, gv1)
    qv = (qv0, qv1)
    gsem = (gsem0, gsem1)
    qsem = (qsem0, qsem1)

    @plsc.parallel_loop(0, ZCH, unroll=4)
    def _zero_row(r):
        for g in range(D // 16):
            zbuf[r, pl.ds(16 * g, 16)] = jnp.zeros((16,), jnp.float32)
    for k in range(pl.cdiv(NZ, NS)):
        t = s + k * NS
        if (k + 1) * NS <= NZ:
            pltpu.sync_copy(zbuf, aggr_sh.at[pl.ds(t * ZCH, ZCH)])
        else:
            @pl.when(t < NZ)
            def _():
                pltpu.sync_copy(zbuf, aggr_sh.at[pl.ds(t * ZCH, ZCH)])
    plsc.subcore_barrier()

    def _start_loads(j, b):
        base = ebase + j * C
        pltpu.sync_copy(col_hbm.at[pl.ds(base, C)], colv[b])
        pltpu.sync_copy(row_hbm.at[pl.ds(base, C)], rowv[b])
        pltpu.async_copy(g_hbm.at[colv[b]], gv[b], gsem[b])
        pltpu.async_copy(q_hbm.at[pl.ds(base, C)], qv[b], qsem[b])

    _start_loads(0, 0)
    _start_loads(1, 1)

    def _round(i, _):
        for b in range(2):
            j = 2 * i + b
            pltpu.make_async_copy(g_hbm.at[colv[b]], gv[b], gsem[b]).wait()
            pltpu.make_async_copy(q_hbm.at[pl.ds(0, C)], qv[b], qsem[b]).wait()

            @plsc.parallel_loop(0, C, unroll=4)
            def _edge(r):
                for t in range(H // 16):
                    qw = qv[b][r, pl.ds(16 * t, 16)]
                    for half in range(2):
                        base = H * half + 16 * t
                        gw = gv[b][r, pl.ds(base, 16)]
                        xval = _lo_f32(gw)
                        pval = _hi_f32(gw)
                        qval = _lo_f32(qw) if half == 0 else _hi_f32(qw)
                        a = 1.0 / (1.0 + jnp.exp(pval + qval))
                        vv[r, pl.ds(base, 16)] = xval * a

            pltpu.sync_copy(vv, aggr_sh.at[rowv[b]], add=True)

            _start_loads(lax.rem(j + 2, NCHUNK), b)
        return _
    lax.fori_loop(0, NCHUNK // 2, _round, None)
    for b in range(2):
        pltpu.make_async_copy(g_hbm.at[colv[b]], gv[b], gsem[b]).wait()
        pltpu.make_async_copy(q_hbm.at[pl.ds(0, C)], qv[b], qsem[b]).wait()
    plsc.subcore_barrier()

    for k in range(pl.cdiv(NZ, NS)):
        t = s + k * NS

        def _copy_out(t=t):
            pltpu.sync_copy(aggr_sh.at[pl.ds(t * ZCH, ZCH)], zbuf)
            pltpu.sync_copy(zbuf, out_hbm.at[c, pl.ds(t * ZCH, ZCH)])
        if (k + 1) * NS <= NZ:
            _copy_out()
        else:
            pl.when(t < NZ)(_copy_out)


_sc_scatter = functools.partial(
    pl.kernel,
    out_type=jax.ShapeDtypeStruct((NC, N, D), jnp.float32),
    mesh=plsc.VectorSubcoreMesh(core_axis_name="c", subcore_axis_name="s"),
    scratch_types=[
        pltpu.VMEM((C,), jnp.int32),
        pltpu.VMEM((C,), jnp.int32),
        pltpu.VMEM((C,), jnp.int32),
        pltpu.VMEM((C,), jnp.int32),
        pltpu.VMEM((C, D), jnp.int32),
        pltpu.VMEM((C, D), jnp.int32),
        pltpu.VMEM((C, H), jnp.int32),
        pltpu.VMEM((C, H), jnp.int32),
        pltpu.VMEM((C, D), jnp.float32),
        pltpu.VMEM((ZCH, D), jnp.float32),
        pltpu.VMEM_SHARED((N, D), jnp.float32),
        pltpu.SemaphoreType.DMA,
        pltpu.SemaphoreType.DMA,
        pltpu.SemaphoreType.DMA,
        pltpu.SemaphoreType.DMA,
    ],
)(_sc_body)


def kernel(x, edge_index, edge_attr, W_node_w, W_node_b, W_neigh_w, W_neigh_b,
           W_att_w, W_att_b):
    row = edge_index[0]
    col = edge_index[1]
    W1t = W_att_w[:, :D].T
    W2t = W_att_w[:, D:].T

    G = pl.pallas_call(
        _g_kernel,
        out_shape=jax.ShapeDtypeStruct((N, D), jnp.int32),
    )(x, W1t)

    QB = 8000
    Q = pl.pallas_call(
        _q_kernel,
        grid=(E // QB,),
        in_specs=[
            pl.BlockSpec((QB, ED), lambda i: (i, 0)),
            pl.BlockSpec((ED, D), lambda i: (0, 0)),
            pl.BlockSpec((1, D), lambda i: (0, 0)),
        ],
        out_specs=pl.BlockSpec((QB, H), lambda i: (i, 0)),
        out_shape=jax.ShapeDtypeStruct((E, H), jnp.int32),
    )(edge_attr, W2t, W_att_b.reshape(1, D))

    aggr_parts = _sc_scatter(G, Q, col, row)

    out = pl.pallas_call(
        _out_kernel,
        out_shape=jax.ShapeDtypeStruct((N, D), jnp.float32),
    )(x, aggr_parts[0], aggr_parts[1], W_node_w.T, W_neigh_w.T,
      (W_node_b + W_neigh_b).reshape(1, D))
    return out


# speedup vs baseline: 1.3584x; 1.0399x over previous
import functools

import jax
import jax.numpy as jnp
import numpy as np
from jax import lax
from jax.experimental import pallas as pl
from jax.experimental.pallas import tpu as pltpu
from jax.experimental.pallas import tpu_sc as plsc

N = 10000
E = 320000
D = 128
ED = 16
H = D // 2

NC = 2
NS = 16
NW = NC * NS
C = 72
NCHUNK = 140
EPW = NCHUNK * C
EP = NW * EPW
ZCH = 40
NZ = N // ZCH

QB = EP // 40
GB = 256

_HI = np.int32(-65536)


def _pack2(lo_f32, hi_f32):
    lob = lax.bitcast_convert_type(
        lo_f32.astype(jnp.bfloat16).astype(jnp.float32), jnp.uint32)
    hib = lax.bitcast_convert_type(
        hi_f32.astype(jnp.bfloat16).astype(jnp.float32), jnp.uint32)
    return lax.bitcast_convert_type((lob >> 16) | (hib & jnp.uint32(0xFFFF0000)),
                                    jnp.int32)


def _gq_kernel(x_ref, w1t_ref, ea_ref, w2t_ref, b_ref, g_ref, q_ref):
    i = pl.program_id(0)
    p = -jnp.dot(x_ref[...], w1t_ref[...], preferred_element_type=jnp.float32)
    g_ref[...] = _pack2(x_ref[...], p)
    q = -(jnp.dot(ea_ref[...], w2t_ref[...],
                  preferred_element_type=jnp.float32) + b_ref[...])
    ridx = i * QB + lax.broadcasted_iota(jnp.int32, (QB, 1), 0)
    q = jnp.where(ridx >= E, jnp.float32(3e4), q)
    q_ref[...] = _pack2(q[:, :H], q[:, H:])


def _out_kernel(x_ref, a0_ref, a1_ref, wnt_ref, wwt_ref, b_ref, o_ref):
    acc = jnp.dot(x_ref[...], wnt_ref[...], preferred_element_type=jnp.float32)
    aggr = a0_ref[...] + a1_ref[...]
    acc += jnp.dot(aggr, wwt_ref[...], preferred_element_type=jnp.float32)
    o_ref[...] = jnp.tanh(acc + b_ref[...])


def _lo_f32(w):
    return lax.bitcast_convert_type(lax.shift_left(w, 16), jnp.float32)


def _hi_f32(w):
    return lax.bitcast_convert_type(lax.bitwise_and(w, _HI), jnp.float32)


def _sc_body(g_hbm, q_hbm, col_hbm, row_hbm, out_hbm,
             colv0, colv1, rowv0, rowv1, gv0, gv1, qv0, qv1, vv,
             aggr_sh, gsem0, gsem1, qsem0, qsem1):
    c = lax.axis_index("c")
    s = lax.axis_index("s")
    wid = c * NS + s
    ebase = wid * EPW
    colv = (colv0, colv1)
    rowv = (rowv0, rowv1)
    gv = (gv0, gv1)
    qv = (qv0, qv1)
    gsem = (gsem0, gsem1)
    qsem = (qsem0, qsem1)

    @plsc.parallel_loop(0, ZCH, unroll=4)
    def _zero_row(r):
        for g in range(D // 16):
            vv[r, pl.ds(16 * g, 16)] = jnp.zeros((16,), jnp.float32)
    for k in range(pl.cdiv(NZ, NS)):
        t = s + k * NS
        if (k + 1) * NS <= NZ:
            pltpu.sync_copy(vv.at[pl.ds(0, ZCH)],
                            aggr_sh.at[pl.ds(t * ZCH, ZCH)])
        else:
            @pl.when(t < NZ)
            def _():
                pltpu.sync_copy(vv.at[pl.ds(0, ZCH)],
                                aggr_sh.at[pl.ds(t * ZCH, ZCH)])
    plsc.subcore_barrier()

    def _start_loads(j, b):
        base = ebase + j * C
        pltpu.sync_copy(col_hbm.at[pl.ds(base, C)], colv[b])
        pltpu.sync_copy(row_hbm.at[pl.ds(base, C)], rowv[b])
        pltpu.async_copy(g_hbm.at[colv[b]], gv[b], gsem[b])
        pltpu.async_copy(q_hbm.at[pl.ds(base, C)], qv[b], qsem[b])

    _start_loads(0, 0)
    _start_loads(1, 1)

    def _round(i, _):
        for b in range(2):
            j = 2 * i + b
            pltpu.make_async_copy(g_hbm.at[colv[b]], gv[b], gsem[b]).wait()
            pltpu.make_async_copy(q_hbm.at[pl.ds(0, C)], qv[b], qsem[b]).wait()

            @plsc.parallel_loop(0, C, unroll=4)
            def _edge(r):
                for t in range(H // 16):
                    qw = qv[b][r, pl.ds(16 * t, 16)]
                    for half in range(2):
                        base = H * half + 16 * t
                        gw = gv[b][r, pl.ds(base, 16)]
                        xval = _lo_f32(gw)
                        pval = _hi_f32(gw)
                        qval = _lo_f32(qw) if half == 0 else _hi_f32(qw)
                        a = 1.0 / (1.0 + jnp.exp(pval + qval))
                        vv[r, pl.ds(base, 16)] = xval * a

            pltpu.sync_copy(vv, aggr_sh.at[rowv[b]], add=True)

            _start_loads(lax.rem(j + 2, NCHUNK), b)
        return _
    lax.fori_loop(0, NCHUNK // 2, _round, None)
    for b in range(2):
        pltpu.make_async_copy(g_hbm.at[colv[b]], gv[b], gsem[b]).wait()
        pltpu.make_async_copy(q_hbm.at[pl.ds(0, C)], qv[b], qsem[b]).wait()
    plsc.subcore_barrier()

    for k in range(pl.cdiv(NZ, NS)):
        t = s + k * NS

        def _copy_out(t=t):
            pltpu.sync_copy(aggr_sh.at[pl.ds(t * ZCH, ZCH)],
                            vv.at[pl.ds(0, ZCH)])
            pltpu.sync_copy(vv.at[pl.ds(0, ZCH)],
                            out_hbm.at[c, pl.ds(t * ZCH, ZCH)])
        if (k + 1) * NS <= NZ:
            _copy_out()
        else:
            pl.when(t < NZ)(_copy_out)


_sc_scatter = functools.partial(
    pl.kernel,
    out_type=jax.ShapeDtypeStruct((NC, N, D), jnp.float32),
    mesh=plsc.VectorSubcoreMesh(core_axis_name="c", subcore_axis_name="s"),
    scratch_types=[
        pltpu.VMEM((C,), jnp.int32),
        pltpu.VMEM((C,), jnp.int32),
        pltpu.VMEM((C,), jnp.int32),
        pltpu.VMEM((C,), jnp.int32),
        pltpu.VMEM((C, D), jnp.int32),
        pltpu.VMEM((C, D), jnp.int32),
        pltpu.VMEM((C, H), jnp.int32),
        pltpu.VMEM((C, H), jnp.int32),
        pltpu.VMEM((C, D), jnp.float32),
        pltpu.VMEM_SHARED((N, D), jnp.float32),
        pltpu.SemaphoreType.DMA,
        pltpu.SemaphoreType.DMA,
        pltpu.SemaphoreType.DMA,
        pltpu.SemaphoreType.DMA,
    ],
)(_sc_body)


def kernel(x, edge_index, edge_attr, W_node_w, W_node_b, W_neigh_w, W_neigh_b,
           W_att_w, W_att_b):
    pad = jnp.zeros((EP - E,), jnp.int32)
    row = jnp.concatenate([edge_index[0], pad])
    col = jnp.concatenate([edge_index[1], pad])
    ea_pad = jnp.concatenate(
        [edge_attr, jnp.zeros((EP - E, ED), jnp.float32)])
    W1t = W_att_w[:, :D].T
    W2t = W_att_w[:, D:].T

    G, Q = pl.pallas_call(
        _gq_kernel,
        grid=(EP // QB,),
        in_specs=[
            pl.BlockSpec((GB, D), lambda i: (i, 0)),
            pl.BlockSpec((D, D), lambda i: (0, 0)),
            pl.BlockSpec((QB, ED), lambda i: (i, 0)),
            pl.BlockSpec((ED, D), lambda i: (0, 0)),
            pl.BlockSpec((1, D), lambda i: (0, 0)),
        ],
        out_specs=[
            pl.BlockSpec((GB, D), lambda i: (i, 0)),
            pl.BlockSpec((QB, H), lambda i: (i, 0)),
        ],
        out_shape=[
            jax.ShapeDtypeStruct((N, D), jnp.int32),
            jax.ShapeDtypeStruct((EP, H), jnp.int32),
        ],
    )(x, W1t, ea_pad, W2t, W_att_b.reshape(1, D))

    aggr_parts = _sc_scatter(G, Q, col, row)

    out = pl.pallas_call(
        _out_kernel,
        out_shape=jax.ShapeDtypeStruct((N, D), jnp.float32),
    )(x, aggr_parts[0], aggr_parts[1], W_node_w.T, W_neigh_w.T,
      (W_node_b + W_neigh_b).reshape(1, D))
    return out


# speedup vs baseline: 1.4137x; 1.0407x over previous
import functools

import jax
import jax.numpy as jnp
import numpy as np
from jax import lax
from jax.experimental import pallas as pl
from jax.experimental.pallas import tpu as pltpu
from jax.experimental.pallas import tpu_sc as plsc

N = 10000
E = 320000
D = 128
ED = 16
H = D // 2

NC = 2
NS = 16
NW = NC * NS
C = 72
NCHUNK = 140
EPW = NCHUNK * C
EP = NW * EPW
ZCH = 40
NZ = N // ZCH

QB = EP // 40
GB = 256

_HI = np.int32(-65536)


def _pack2(lo_f32, hi_f32):
    lob = lax.bitcast_convert_type(
        lo_f32.astype(jnp.bfloat16).astype(jnp.float32), jnp.uint32)
    hib = lax.bitcast_convert_type(
        hi_f32.astype(jnp.bfloat16).astype(jnp.float32), jnp.uint32)
    return lax.bitcast_convert_type((lob >> 16) | (hib & jnp.uint32(0xFFFF0000)),
                                    jnp.int32)


def _gq_kernel(x_ref, w1t_ref, ea_ref, w2t_ref, b_ref, g_ref, q_ref):
    i = pl.program_id(0)
    p = -jnp.dot(x_ref[...], w1t_ref[...], preferred_element_type=jnp.float32)
    g_ref[...] = _pack2(x_ref[...], p)
    q = -(jnp.dot(ea_ref[...], w2t_ref[...],
                  preferred_element_type=jnp.float32) + b_ref[...])
    ridx = i * QB + lax.broadcasted_iota(jnp.int32, (QB, 1), 0)
    q = jnp.where(ridx >= E, jnp.float32(3e4), q)
    q_ref[...] = _pack2(q[:, :H], q[:, H:])


def _out_kernel(x_ref, a0_ref, a1_ref, wnt_ref, wwt_ref, b_ref, o_ref):
    acc = jnp.dot(x_ref[...], wnt_ref[...], preferred_element_type=jnp.float32)
    aggr = a0_ref[...] + a1_ref[...]
    acc += jnp.dot(aggr, wwt_ref[...], preferred_element_type=jnp.float32)
    o_ref[...] = jnp.tanh(acc + b_ref[...])


def _lo_f32(w):
    return lax.bitcast_convert_type(lax.shift_left(w, 16), jnp.float32)


def _hi_f32(w):
    return lax.bitcast_convert_type(lax.bitwise_and(w, _HI), jnp.float32)


def _sc_body(g_hbm, q_hbm, col_hbm, row_hbm, out_hbm,
             colv0, colv1, rowv0, rowv1, gv0, gv1, qv0, qv1, vv,
             aggr_sh, gsem0, gsem1, qsem0, qsem1):
    c = lax.axis_index("c")
    s = lax.axis_index("s")
    wid = c * NS + s
    ebase = wid * EPW
    colv = (colv0, colv1)
    rowv = (rowv0, rowv1)
    gv = (gv0, gv1)
    qv = (qv0, qv1)
    gsem = (gsem0, gsem1)
    qsem = (qsem0, qsem1)

    @plsc.parallel_loop(0, ZCH, unroll=4)
    def _zero_row(r):
        for g in range(D // 16):
            vv[r, pl.ds(16 * g, 16)] = jnp.zeros((16,), jnp.float32)
    for k in range(pl.cdiv(NZ, NS)):
        t = s + k * NS
        if (k + 1) * NS <= NZ:
            pltpu.sync_copy(vv.at[pl.ds(0, ZCH)],
                            aggr_sh.at[pl.ds(t * ZCH, ZCH)])
        else:
            @pl.when(t < NZ)
            def _():
                pltpu.sync_copy(vv.at[pl.ds(0, ZCH)],
                                aggr_sh.at[pl.ds(t * ZCH, ZCH)])
    plsc.subcore_barrier()

    def _start_loads(j, b):
        base = ebase + j * C
        pltpu.sync_copy(col_hbm.at[pl.ds(base, C)], colv[b])
        pltpu.sync_copy(row_hbm.at[pl.ds(base, C)], rowv[b])
        pltpu.async_copy(g_hbm.at[colv[b]], gv[b], gsem[b])
        pltpu.async_copy(q_hbm.at[pl.ds(base, C)], qv[b], qsem[b])

    _start_loads(0, 0)
    _start_loads(1, 1)

    def _round(i, _):
        for b in range(2):
            j = 2 * i + b
            pltpu.make_async_copy(g_hbm.at[colv[b]], gv[b], gsem[b]).wait()
            pltpu.make_async_copy(q_hbm.at[pl.ds(0, C)], qv[b], qsem[b]).wait()

            @plsc.parallel_loop(0, C, unroll=4)
            def _edge(r):
                for t in range(H // 16):
                    qw = qv[b][r, pl.ds(16 * t, 16)]
                    for half in range(2):
                        base = H * half + 16 * t
                        gw = gv[b][r, pl.ds(base, 16)]
                        xval = _lo_f32(gw)
                        pval = _hi_f32(gw)
                        qval = _lo_f32(qw) if half == 0 else _hi_f32(qw)
                        a = 1.0 / (1.0 + jnp.exp(pval + qval))
                        vv[r, pl.ds(base, 16)] = xval * a

            pltpu.sync_copy(vv, aggr_sh.at[rowv[b]], add=True)

            _start_loads(lax.rem(j + 2, NCHUNK), b)
        return _
    lax.fori_loop(0, NCHUNK // 2, _round, None)
    for b in range(2):
        pltpu.make_async_copy(g_hbm.at[colv[b]], gv[b], gsem[b]).wait()
        pltpu.make_async_copy(q_hbm.at[pl.ds(0, C)], qv[b], qsem[b]).wait()
    plsc.subcore_barrier()

    for k in range(pl.cdiv(NZ, NS)):
        t = s + k * NS

        def _copy_out(t=t):
            pltpu.sync_copy(aggr_sh.at[pl.ds(t * ZCH, ZCH)],
                            vv.at[pl.ds(0, ZCH)])
            pltpu.sync_copy(vv.at[pl.ds(0, ZCH)],
                            out_hbm.at[c, pl.ds(t * ZCH, ZCH)])
        if (k + 1) * NS <= NZ:
            _copy_out()
        else:
            pl.when(t < NZ)(_copy_out)


_sc_scatter = functools.partial(
    pl.kernel,
    out_type=jax.ShapeDtypeStruct((NC, N, D), jnp.float32),
    mesh=plsc.VectorSubcoreMesh(core_axis_name="c", subcore_axis_name="s"),
    scratch_types=[
        pltpu.VMEM((C,), jnp.int32),
        pltpu.VMEM((C,), jnp.int32),
        pltpu.VMEM((C,), jnp.int32),
        pltpu.VMEM((C,), jnp.int32),
        pltpu.VMEM((C, D), jnp.int32),
        pltpu.VMEM((C, D), jnp.int32),
        pltpu.VMEM((C, H), jnp.int32),
        pltpu.VMEM((C, H), jnp.int32),
        pltpu.VMEM((C, D), jnp.float32),
        pltpu.VMEM_SHARED((N, D), jnp.float32),
        pltpu.SemaphoreType.DMA,
        pltpu.SemaphoreType.DMA,
        pltpu.SemaphoreType.DMA,
        pltpu.SemaphoreType.DMA,
    ],
)(_sc_body)


def kernel(x, edge_index, edge_attr, W_node_w, W_node_b, W_neigh_w, W_neigh_b,
           W_att_w, W_att_b):
    row = jnp.pad(edge_index[0], (0, EP - E))
    col = jnp.pad(edge_index[1], (0, EP - E))
    W1t = W_att_w[:, :D].T
    W2t = W_att_w[:, D:].T

    G, Q = pl.pallas_call(
        _gq_kernel,
        grid=(EP // QB,),
        in_specs=[
            pl.BlockSpec((GB, D), lambda i: (i, 0)),
            pl.BlockSpec((D, D), lambda i: (0, 0)),
            pl.BlockSpec((QB, ED), lambda i: (i, 0)),
            pl.BlockSpec((ED, D), lambda i: (0, 0)),
            pl.BlockSpec((1, D), lambda i: (0, 0)),
        ],
        out_specs=[
            pl.BlockSpec((GB, D), lambda i: (i, 0)),
            pl.BlockSpec((QB, H), lambda i: (i, 0)),
        ],
        out_shape=[
            jax.ShapeDtypeStruct((N, D), jnp.int32),
            jax.ShapeDtypeStruct((EP, H), jnp.int32),
        ],
    )(x, W1t, edge_attr, W2t, W_att_b.reshape(1, D))

    aggr_parts = _sc_scatter(G, Q, col, row)

    out = pl.pallas_call(
        _out_kernel,
        out_shape=jax.ShapeDtypeStruct((N, D), jnp.float32),
    )(x, aggr_parts[0], aggr_parts[1], W_node_w.T, W_neigh_w.T,
      (W_node_b + W_neigh_b).reshape(1, D))
    return out
